# Initial kernel scaffold; baseline (speedup 1.0000x reference)
import functools

import jax
import jax.numpy as jnp
from jax import lax
from jax.experimental import pallas as pl
from jax.experimental.pallas import tpu as pltpu
from jax.experimental.pallas import tpu_sc as plsc

N = 10000
NPAD = 10240
CHUNK = 128
NCORES = 2
NSUB = 16
NW = NCORES * NSUB
ROWS_PER_SUB = NPAD // NSUB
ZCOPIES = ROWS_PER_SUB // CHUNK


def _make_agg(D, epad):
    per_tile = epad // NW
    n_chunks = per_tile // CHUNK
    mesh = plsc.VectorSubcoreMesh(core_axis_name="c", subcore_axis_name="s")

    @functools.partial(
        pl.kernel,
        out_type=jax.ShapeDtypeStruct((NCORES, NPAD, D), jnp.float32),
        mesh=mesh,
        scratch_types=[
            pltpu.VMEM((CHUNK,), jnp.int32),
            pltpu.VMEM((CHUNK,), jnp.int32),
            pltpu.VMEM((CHUNK, D), jnp.float32),
            pltpu.VMEM_SHARED((NPAD, D), jnp.float32),
            pltpu.SemaphoreType.DMA,
        ],
    )
    def agg(table_hbm, src_hbm, dst_hbm, out_hbm, src_v, dst_v, rows_v, acc_sh, sem):
        c = lax.axis_index("c")
        s = lax.axis_index("s")
        wid = c * NSUB + s

        @pl.loop(0, CHUNK)
        def _zero_rows(i):
            @pl.loop(0, D, step=16)
            def _zero_lanes(j):
                rows_v[i, pl.ds(j, 16)] = jnp.zeros((16,), jnp.float32)

        @pl.loop(0, ZCOPIES)
        def _zero_acc(k):
            pltpu.sync_copy(rows_v, acc_sh.at[pl.ds(s * ROWS_PER_SUB + k * CHUNK, CHUNK)])

        plsc.subcore_barrier()

        base = wid * per_tile

        @pl.loop(0, n_chunks)
        def _edges(t):
            off = base + t * CHUNK
            pltpu.sync_copy(src_hbm.at[pl.ds(off, CHUNK)], src_v)
            pltpu.sync_copy(dst_hbm.at[pl.ds(off, CHUNK)], dst_v)
            pltpu.async_copy(table_hbm.at[src_v], rows_v, sem).wait()
            pltpu.sync_copy(rows_v, acc_sh.at[dst_v], add=True)

        plsc.subcore_barrier()
        pltpu.sync_copy(
            acc_sh.at[pl.ds(s * ROWS_PER_SUB, ROWS_PER_SUB)],
            out_hbm.at[c, pl.ds(s * ROWS_PER_SUB, ROWS_PER_SUB)],
        )

    return agg


def _prep_body(degp_ref, x_ref, dis_ref, xt_ref):
    deg = degp_ref[0, :, 0:1] + degp_ref[1, :, 0:1] + 1.0
    dis = lax.rsqrt(deg)
    dis_ref[...] = dis
    xt_ref[...] = x_ref[...] * dis


def _mid_body(s1_ref, xt_ref, dis_ref, w1_ref, b1_ref, w2_ref, zt_ref):
    y = (s1_ref[0] + s1_ref[1] + xt_ref[...]) * dis_ref[...]
    h = jnp.dot(y, w1_ref[...], preferred_element_type=jnp.float32,
                precision=lax.Precision.HIGHEST)
    h = jnp.maximum(h + b1_ref[...], 0.0)
    z = jnp.dot(h, w2_ref[...], preferred_element_type=jnp.float32,
                precision=lax.Precision.HIGHEST)
    zt_ref[...] = z * dis_ref[...]


def _out_body(s2_ref, zt_ref, dis_ref, b2_ref, o_ref):
    z = (s2_ref[0] + s2_ref[1] + zt_ref[...]) * dis_ref[...]
    o = z[:, :40] + b2_ref[...]
    m = jnp.max(o, axis=1, keepdims=True)
    o = o - m
    o_ref[...] = o - jnp.log(jnp.sum(jnp.exp(o), axis=1, keepdims=True))


_R = 2048


def _prep(degp, x_p):
    grid = NPAD // _R
    return pl.pallas_call(
        _prep_body,
        grid=(grid,),
        in_specs=[
            pl.BlockSpec((NCORES, _R, 16), lambda i: (0, i, 0)),
            pl.BlockSpec((_R, 128), lambda i: (i, 0)),
        ],
        out_specs=[
            pl.BlockSpec((_R, 1), lambda i: (i, 0)),
            pl.BlockSpec((_R, 128), lambda i: (i, 0)),
        ],
        out_shape=[
            jax.ShapeDtypeStruct((NPAD, 1), jnp.float32),
            jax.ShapeDtypeStruct((NPAD, 128), jnp.float32),
        ],
    )(degp, x_p)


def _mid(s1, xt, dis, w1, b1, w2p):
    grid = NPAD // _R
    return pl.pallas_call(
        _mid_body,
        grid=(grid,),
        in_specs=[
            pl.BlockSpec((NCORES, _R, 128), lambda i: (0, i, 0)),
            pl.BlockSpec((_R, 128), lambda i: (i, 0)),
            pl.BlockSpec((_R, 1), lambda i: (i, 0)),
            pl.BlockSpec((128, 256), lambda i: (0, 0)),
            pl.BlockSpec((1, 256), lambda i: (0, 0)),
            pl.BlockSpec((256, 48), lambda i: (0, 0)),
        ],
        out_specs=pl.BlockSpec((_R, 48), lambda i: (i, 0)),
        out_shape=jax.ShapeDtypeStruct((NPAD, 48), jnp.float32),
    )(s1, xt, dis, w1, b1, w2p)


def _out(s2, zt, dis, b2):
    grid = NPAD // _R
    return pl.pallas_call(
        _out_body,
        grid=(grid,),
        in_specs=[
            pl.BlockSpec((NCORES, _R, 48), lambda i: (0, i, 0)),
            pl.BlockSpec((_R, 48), lambda i: (i, 0)),
            pl.BlockSpec((_R, 1), lambda i: (i, 0)),
            pl.BlockSpec((1, 40), lambda i: (0, 0)),
        ],
        out_specs=pl.BlockSpec((_R, 40), lambda i: (i, 0)),
        out_shape=jax.ShapeDtypeStruct((NPAD, 40), jnp.float32),
    )(s2, zt, dis, b2)


def kernel(x, edge_index, W1, b1, W2, b2):
    e = edge_index.shape[1]
    epad = ((e + NW * CHUNK - 1) // (NW * CHUNK)) * (NW * CHUNK)
    src = edge_index[0].astype(jnp.int32)
    dst = edge_index[1].astype(jnp.int32)
    pad = jnp.full((epad - e,), N, jnp.int32)
    src_p = jnp.concatenate([src, pad])
    dst_p = jnp.concatenate([dst, pad])
    x_p = jnp.pad(x, ((0, NPAD - N), (0, 0)))

    ones_t = jnp.ones((NPAD, 16), jnp.float32)
    degp = _make_agg(16, epad)(ones_t, dst_p, dst_p)
    dis, xt = _prep(degp, x_p)
    s1 = _make_agg(128, epad)(xt, src_p, dst_p)
    w2p = jnp.pad(W2, ((0, 0), (0, 48 - W2.shape[1])))
    zt = _mid(s1, xt, dis, W1, b1.reshape(1, -1), w2p)
    s2 = _make_agg(48, epad)(zt, src_p, dst_p)
    o = _out(s2, zt, dis, b2.reshape(1, -1))
    return o[:N]


# speedup vs baseline: 12.3544x; 12.3544x over previous
import functools

import jax
import jax.numpy as jnp
from jax import lax
from jax.experimental import pallas as pl
from jax.experimental.pallas import tpu as pltpu
from jax.experimental.pallas import tpu_sc as plsc

N = 10000
NPAD = 10240
CHUNK = 128
NCORES = 2
NSUB = 16
NW = NCORES * NSUB
ROWS_PER_SUB = NPAD // NSUB
ZCOPIES = ROWS_PER_SUB // CHUNK


def _make_agg(D, epad):
    per_tile = epad // NW
    n_chunks = per_tile // CHUNK
    mesh = plsc.VectorSubcoreMesh(core_axis_name="c", subcore_axis_name="s")

    @functools.partial(
        pl.kernel,
        out_type=jax.ShapeDtypeStruct((NCORES, NPAD, D), jnp.float32),
        mesh=mesh,
        compiler_params=pltpu.CompilerParams(use_tc_tiling_on_sc=False),
        scratch_types=[
            pltpu.VMEM((CHUNK,), jnp.int32),
            pltpu.VMEM((CHUNK,), jnp.int32),
            pltpu.VMEM((CHUNK, D), jnp.float32),
            pltpu.VMEM_SHARED((NPAD, D), jnp.float32),
            pltpu.SemaphoreType.DMA,
        ],
    )
    def agg(table_hbm, src_hbm, dst_hbm, out_hbm, src_v, dst_v, rows_v, acc_sh, sem):
        c = lax.axis_index("c")
        s = lax.axis_index("s")
        wid = c * NSUB + s

        @pl.loop(0, CHUNK)
        def _zero_rows(i):
            @pl.loop(0, D, step=16)
            def _zero_lanes(j):
                rows_v[i, pl.ds(j, 16)] = jnp.zeros((16,), jnp.float32)

        @pl.loop(0, ZCOPIES)
        def _zero_acc(k):
            pltpu.sync_copy(rows_v, acc_sh.at[pl.ds(s * ROWS_PER_SUB + k * CHUNK, CHUNK)])

        plsc.subcore_barrier()

        base = wid * per_tile

        @pl.loop(0, n_chunks)
        def _edges(t):
            off = base + t * CHUNK
            pltpu.sync_copy(src_hbm.at[pl.ds(off, CHUNK)], src_v)
            pltpu.sync_copy(dst_hbm.at[pl.ds(off, CHUNK)], dst_v)
            pltpu.async_copy(table_hbm.at[src_v], rows_v, sem).wait()
            pltpu.sync_copy(rows_v, acc_sh.at[dst_v], add=True)

        plsc.subcore_barrier()
        pltpu.sync_copy(
            acc_sh.at[pl.ds(s * ROWS_PER_SUB, ROWS_PER_SUB)],
            out_hbm.at[c, pl.ds(s * ROWS_PER_SUB, ROWS_PER_SUB)],
        )

    return agg


def _prep_body(degp_ref, x_ref, dis_ref, xt_ref):
    deg = degp_ref[0, :, 0:1] + degp_ref[1, :, 0:1] + 1.0
    dis = lax.rsqrt(deg)
    dis_ref[...] = dis
    xt_ref[...] = x_ref[...] * dis


def _mid_body(s1_ref, xt_ref, dis_ref, w1_ref, b1_ref, w2_ref, zt_ref):
    y = (s1_ref[0] + s1_ref[1] + xt_ref[...]) * dis_ref[...]
    h = jnp.dot(y, w1_ref[...], preferred_element_type=jnp.float32,
                precision=lax.Precision.HIGHEST)
    h = jnp.maximum(h + b1_ref[...], 0.0)
    z = jnp.dot(h, w2_ref[...], preferred_element_type=jnp.float32,
                precision=lax.Precision.HIGHEST)
    zt_ref[...] = z * dis_ref[...]


def _out_body(s2_ref, zt_ref, dis_ref, b2_ref, o_ref):
    z = (s2_ref[0] + s2_ref[1] + zt_ref[...]) * dis_ref[...]
    o = z[:, :40] + b2_ref[...]
    m = jnp.max(o, axis=1, keepdims=True)
    o = o - m
    o_ref[...] = o - jnp.log(jnp.sum(jnp.exp(o), axis=1, keepdims=True))


_R = 2048


def _prep(degp, x_p):
    grid = NPAD // _R
    return pl.pallas_call(
        _prep_body,
        grid=(grid,),
        in_specs=[
            pl.BlockSpec((NCORES, _R, 16), lambda i: (0, i, 0)),
            pl.BlockSpec((_R, 128), lambda i: (i, 0)),
        ],
        out_specs=[
            pl.BlockSpec((_R, 1), lambda i: (i, 0)),
            pl.BlockSpec((_R, 128), lambda i: (i, 0)),
        ],
        out_shape=[
            jax.ShapeDtypeStruct((NPAD, 1), jnp.float32),
            jax.ShapeDtypeStruct((NPAD, 128), jnp.float32),
        ],
    )(degp, x_p)


def _mid(s1, xt, dis, w1, b1, w2p):
    grid = NPAD // _R
    return pl.pallas_call(
        _mid_body,
        grid=(grid,),
        in_specs=[
            pl.BlockSpec((NCORES, _R, 128), lambda i: (0, i, 0)),
            pl.BlockSpec((_R, 128), lambda i: (i, 0)),
            pl.BlockSpec((_R, 1), lambda i: (i, 0)),
            pl.BlockSpec((128, 256), lambda i: (0, 0)),
            pl.BlockSpec((1, 256), lambda i: (0, 0)),
            pl.BlockSpec((256, 48), lambda i: (0, 0)),
        ],
        out_specs=pl.BlockSpec((_R, 48), lambda i: (i, 0)),
        out_shape=jax.ShapeDtypeStruct((NPAD, 48), jnp.float32),
    )(s1, xt, dis, w1, b1, w2p)


def _out(s2, zt, dis, b2):
    grid = NPAD // _R
    return pl.pallas_call(
        _out_body,
        grid=(grid,),
        in_specs=[
            pl.BlockSpec((NCORES, _R, 48), lambda i: (0, i, 0)),
            pl.BlockSpec((_R, 48), lambda i: (i, 0)),
            pl.BlockSpec((_R, 1), lambda i: (i, 0)),
            pl.BlockSpec((1, 40), lambda i: (0, 0)),
        ],
        out_specs=pl.BlockSpec((_R, 40), lambda i: (i, 0)),
        out_shape=jax.ShapeDtypeStruct((NPAD, 40), jnp.float32),
    )(s2, zt, dis, b2)


def kernel(x, edge_index, W1, b1, W2, b2):
    e = edge_index.shape[1]
    epad = ((e + NW * CHUNK - 1) // (NW * CHUNK)) * (NW * CHUNK)
    src = edge_index[0].astype(jnp.int32)
    dst = edge_index[1].astype(jnp.int32)
    pad = jnp.full((epad - e,), N, jnp.int32)
    src_p = jnp.concatenate([src, pad])
    dst_p = jnp.concatenate([dst, pad])
    x_p = jnp.pad(x, ((0, NPAD - N), (0, 0)))

    ones_t = jnp.ones((NPAD, 16), jnp.float32)
    degp = _make_agg(16, epad)(ones_t, dst_p, dst_p)
    dis, xt = _prep(degp, x_p)
    s1 = _make_agg(128, epad)(xt, src_p, dst_p)
    w2p = jnp.pad(W2, ((0, 0), (0, 48 - W2.shape[1])))
    zt = _mid(s1, xt, dis, W1, b1.reshape(1, -1), w2p)
    s2 = _make_agg(48, epad)(zt, src_p, dst_p)
    o = _out(s2, zt, dis, b2.reshape(1, -1))
    return o[:N]


# speedup vs baseline: 16.3714x; 1.3252x over previous
import functools

import jax
import jax.numpy as jnp
from jax import lax
from jax.experimental import pallas as pl
from jax.experimental.pallas import tpu as pltpu
from jax.experimental.pallas import tpu_sc as plsc

N = 10000
NPAD = 10240
NCORES = 2
NSUB = 16
NW = NCORES * NSUB
ROWS_PER_SUB = NPAD // NSUB
BIG = 128
KROWS = 1
NB = 80
EPAD = NW * BIG * NB


def _make_agg(D):
    mesh = plsc.VectorSubcoreMesh(core_axis_name="c", subcore_axis_name="s")

    @functools.partial(
        pl.kernel,
        out_type=jax.ShapeDtypeStruct((NCORES, NPAD, D), jnp.float32),
        mesh=mesh,
        compiler_params=pltpu.CompilerParams(use_tc_tiling_on_sc=False),
        scratch_types=[
            pltpu.VMEM((NB // 2, 128), jnp.int32),
            pltpu.VMEM((NB // 2, 128), jnp.int32),
            pltpu.VMEM((BIG, D), jnp.float32),
            pltpu.VMEM((BIG, D), jnp.float32),
            pltpu.VMEM_SHARED((NPAD, D), jnp.float32),
            pltpu.SemaphoreType.DMA,
            pltpu.SemaphoreType.DMA,
            pltpu.SemaphoreType.DMA,
            pltpu.SemaphoreType.DMA,
        ],
    )
    def agg(table_hbm, src_hbm, dst_hbm, out_hbm,
            src_v, dst_v, rows0, rows1, acc_sh, g0, g1, s0, s1):
        c = lax.axis_index("c")
        s = lax.axis_index("s")
        wid = c * NSUB + s
        rows = (rows0, rows1)
        gsem = (g0, g1)
        ssem = (s0, s1)
        nbh = NB // 2

        @pl.loop(0, BIG)
        def _zero_rows(i):
            @pl.loop(0, D, step=16)
            def _zero_lanes(j):
                rows0[i, pl.ds(j, 16)] = jnp.zeros((16,), jnp.float32)

        @pl.loop(0, ROWS_PER_SUB // BIG)
        def _zero_acc(k):
            pltpu.sync_copy(rows0, acc_sh.at[pl.ds(s * ROWS_PER_SUB + k * BIG, BIG)])

        if ROWS_PER_SUB % BIG != 0:
            pltpu.sync_copy(
                rows0.at[pl.ds(0, ROWS_PER_SUB % BIG)],
                acc_sh.at[pl.ds(s * ROWS_PER_SUB + (ROWS_PER_SUB // BIG) * BIG,
                                ROWS_PER_SUB % BIG)])

        plsc.subcore_barrier()

        for h in range(2):
            pltpu.sync_copy(src_hbm.at[pl.ds((wid * 2 + h) * nbh, nbh)], src_v)
            pltpu.sync_copy(dst_hbm.at[pl.ds((wid * 2 + h) * nbh, nbh)], dst_v)

            pltpu.async_copy(table_hbm.at[src_v.at[0]], rows0, g0)
            pltpu.async_copy(table_hbm.at[src_v.at[1]], rows1, g1)

            @pl.loop(0, nbh // 2 - 1)
            def _edges(t):
                j = t * 2
                for b in range(2):
                    pltpu.make_async_copy(table_hbm.at[src_v.at[j + b]],
                                          rows[b], gsem[b]).wait()
                    pltpu.async_copy(rows[b], acc_sh.at[dst_v.at[j + b]],
                                     ssem[b], add=True)
                    pltpu.make_async_copy(rows[b], acc_sh.at[dst_v.at[j + b]],
                                          ssem[b]).wait()
                    pltpu.async_copy(table_hbm.at[src_v.at[j + 2 + b]],
                                     rows[b], gsem[b])

            for b in range(2):
                pltpu.make_async_copy(table_hbm.at[src_v.at[nbh - 2 + b]],
                                      rows[b], gsem[b]).wait()
                pltpu.async_copy(rows[b], acc_sh.at[dst_v.at[nbh - 2 + b]],
                                 ssem[b], add=True)
                pltpu.make_async_copy(rows[b], acc_sh.at[dst_v.at[nbh - 2 + b]],
                                      ssem[b]).wait()

        plsc.subcore_barrier()
        pltpu.sync_copy(
            acc_sh.at[pl.ds(s * ROWS_PER_SUB, ROWS_PER_SUB)],
            out_hbm.at[c, pl.ds(s * ROWS_PER_SUB, ROWS_PER_SUB)],
        )

    return agg


def _make_deg():
    D = 16
    mesh = plsc.VectorSubcoreMesh(core_axis_name="c", subcore_axis_name="s")

    @functools.partial(
        pl.kernel,
        out_type=jax.ShapeDtypeStruct((NCORES, NPAD, D), jnp.float32),
        mesh=mesh,
        compiler_params=pltpu.CompilerParams(use_tc_tiling_on_sc=False),
        scratch_types=[
            pltpu.VMEM((NB, 128), jnp.int32),
            pltpu.VMEM((BIG, D), jnp.float32),
            pltpu.VMEM((BIG, D), jnp.float32),
            pltpu.VMEM_SHARED((NPAD, D), jnp.float32),
            pltpu.SemaphoreType.DMA,
        ],
    )
    def deg(dst_hbm, out_hbm, dst_v, ones_v, zeros_v, acc_sh, sem):
        c = lax.axis_index("c")
        s = lax.axis_index("s")
        wid = c * NSUB + s

        pltpu.sync_copy(dst_hbm.at[pl.ds(wid * NB, NB)], dst_v)

        @pl.loop(0, BIG)
        def _fill(i):
            @pl.loop(0, D, step=16)
            def _lanes(j):
                ones_v[i, pl.ds(j, 16)] = jnp.ones((16,), jnp.float32)
                zeros_v[i, pl.ds(j, 16)] = jnp.zeros((16,), jnp.float32)

        @pl.loop(0, ROWS_PER_SUB // BIG)
        def _zero_acc(k):
            pltpu.sync_copy(zeros_v, acc_sh.at[pl.ds(s * ROWS_PER_SUB + k * BIG, BIG)])

        if ROWS_PER_SUB % BIG != 0:
            pltpu.sync_copy(
                zeros_v.at[pl.ds(0, ROWS_PER_SUB % BIG)],
                acc_sh.at[pl.ds(s * ROWS_PER_SUB + (ROWS_PER_SUB // BIG) * BIG,
                                ROWS_PER_SUB % BIG)])

        plsc.subcore_barrier()

        @pl.loop(0, NB)
        def _fire(j):
            pltpu.async_copy(ones_v, acc_sh.at[dst_v.at[j]], sem, add=True)

        @pl.loop(0, NB)
        def _drain(j):
            pltpu.make_async_copy(ones_v, acc_sh.at[dst_v.at[j]], sem).wait()

        plsc.subcore_barrier()
        pltpu.sync_copy(
            acc_sh.at[pl.ds(s * ROWS_PER_SUB, ROWS_PER_SUB)],
            out_hbm.at[c, pl.ds(s * ROWS_PER_SUB, ROWS_PER_SUB)],
        )

    return deg


def _prep_body(degp_ref, x_ref, dis_ref, xt_ref):
    deg = degp_ref[0, :, 0:1] + degp_ref[1, :, 0:1] + 1.0
    dis = lax.rsqrt(deg)
    dis_ref[...] = dis
    xt_ref[...] = x_ref[...] * dis


def _mid_body(s1_ref, xt_ref, dis_ref, w1_ref, b1_ref, w2_ref, zt_ref):
    y = (s1_ref[0] + s1_ref[1] + xt_ref[...]) * dis_ref[...]
    h = jnp.dot(y, w1_ref[...], preferred_element_type=jnp.float32,
                precision=lax.Precision.HIGHEST)
    h = jnp.maximum(h + b1_ref[...], 0.0)
    z = jnp.dot(h, w2_ref[...], preferred_element_type=jnp.float32,
                precision=lax.Precision.HIGHEST)
    zt_ref[...] = z * dis_ref[...]


def _out_body(s2_ref, zt_ref, dis_ref, b2_ref, o_ref):
    z = (s2_ref[0] + s2_ref[1] + zt_ref[...]) * dis_ref[...]
    o = z[:, :40] + b2_ref[...]
    m = jnp.max(o, axis=1, keepdims=True)
    o = o - m
    o_ref[...] = o - jnp.log(jnp.sum(jnp.exp(o), axis=1, keepdims=True))


_R = 2048


def _prep(degp, x_p):
    grid = NPAD // _R
    return pl.pallas_call(
        _prep_body,
        grid=(grid,),
        in_specs=[
            pl.BlockSpec((NCORES, _R, 16), lambda i: (0, i, 0)),
            pl.BlockSpec((_R, 128), lambda i: (i, 0)),
        ],
        out_specs=[
            pl.BlockSpec((_R, 1), lambda i: (i, 0)),
            pl.BlockSpec((_R, 128), lambda i: (i, 0)),
        ],
        out_shape=[
            jax.ShapeDtypeStruct((NPAD, 1), jnp.float32),
            jax.ShapeDtypeStruct((NPAD, 128), jnp.float32),
        ],
    )(degp, x_p)


def _mid(s1, xt, dis, w1, b1, w2p):
    grid = NPAD // _R
    return pl.pallas_call(
        _mid_body,
        grid=(grid,),
        in_specs=[
            pl.BlockSpec((NCORES, _R, 128), lambda i: (0, i, 0)),
            pl.BlockSpec((_R, 128), lambda i: (i, 0)),
            pl.BlockSpec((_R, 1), lambda i: (i, 0)),
            pl.BlockSpec((128, 256), lambda i: (0, 0)),
            pl.BlockSpec((1, 256), lambda i: (0, 0)),
            pl.BlockSpec((256, 48), lambda i: (0, 0)),
        ],
        out_specs=pl.BlockSpec((_R, 48), lambda i: (i, 0)),
        out_shape=jax.ShapeDtypeStruct((NPAD, 48), jnp.float32),
    )(s1, xt, dis, w1, b1, w2p)


def _out(s2, zt, dis, b2):
    grid = NPAD // _R
    return pl.pallas_call(
        _out_body,
        grid=(grid,),
        in_specs=[
            pl.BlockSpec((NCORES, _R, 48), lambda i: (0, i, 0)),
            pl.BlockSpec((_R, 48), lambda i: (i, 0)),
            pl.BlockSpec((_R, 1), lambda i: (i, 0)),
            pl.BlockSpec((1, 40), lambda i: (0, 0)),
        ],
        out_specs=pl.BlockSpec((_R, 40), lambda i: (i, 0)),
        out_shape=jax.ShapeDtypeStruct((NPAD, 40), jnp.float32),
    )(s2, zt, dis, b2)


def kernel(x, edge_index, W1, b1, W2, b2):
    e = edge_index.shape[1]
    src = edge_index[0].astype(jnp.int32)
    dst = edge_index[1].astype(jnp.int32)
    pad = jnp.full((EPAD - e,), N, jnp.int32)
    src_p = jnp.concatenate([src, pad]).reshape(NW * NB, 128)
    dst_p = jnp.concatenate([dst, pad]).reshape(NW * NB, 128)
    x_p = jnp.pad(x, ((0, NPAD - N), (0, 0)))

    degp = _make_deg()(dst_p)
    dis, xt = _prep(degp, x_p)
    s1 = _make_agg(128)(xt, src_p, dst_p)
    w2p = jnp.pad(W2, ((0, 0), (0, 48 - W2.shape[1])))
    zt = _mid(s1, xt, dis, W1, b1.reshape(1, -1), w2p)
    s2 = _make_agg(48)(zt, src_p, dst_p)
    o = _out(s2, zt, dis, b2.reshape(1, -1))
    return o[:N]
